# baseline (speedup 1.0000x reference)
import functools

import jax
import jax.numpy as jnp
from jax import lax
from jax.experimental import pallas as pl
from jax.experimental.pallas import tpu as pltpu
from jax.experimental.pallas import tpu_sc as plsc

LEVELS = [10000, 2500, 625, 160, 40]
CH = [3, 32, 64, 128, 256]
CG = [8, 32, 64, 128]
SPIRAL = 9
BATCH = 64
LATENT = 256
NUM_OUT = 10

NW = 32
N_ACT = [32, 32, 32, 8]
RW = [2160, 576, 144, 144]
EPAD = [n * r // SPIRAL for n, r in zip(N_ACT, RW)]
CHUNKS = [(r + 127) // 128 for r in RW]


def _gather_body(n_in, rw, chunks, n_act, sic_ref, h_ref, g_ref, idxr, idxb, stage, sem):
    wid = lax.axis_index("s") * 2 + lax.axis_index("c")

    @pl.when(wid < n_act)
    def _():
        pltpu.sync_copy(sic_ref.at[wid], idxr)

        def b_loop(b, carry):
            base = b * n_in
            for j in range(chunks):
                for k in range(8):
                    sl = pl.ds(k * 16, 16)
                    idxb[j, sl] = idxr[j, sl] + base
            handles = []
            for j in range(chunks):
                handles.append(
                    pltpu.async_copy(
                        h_ref.at[idxb.at[j]],
                        stage.at[pl.ds(j * 128, 128)],
                        sem,
                    )
                )
            for h in handles:
                h.wait()
            pltpu.sync_copy(
                stage.at[pl.ds(0, rw)],
                g_ref.at[b, pl.ds(wid * rw, rw)],
            )
            return carry

        lax.fori_loop(0, BATCH, b_loop, 0)


def _sc_gather(h2, sic_pad, n_in, c, lvl):
    rw, chunks, n_act = RW[lvl], CHUNKS[lvl], N_ACT[lvl]
    mesh = plsc.VectorSubcoreMesh(core_axis_name="c", subcore_axis_name="s")
    body = functools.partial(_gather_body, n_in, rw, chunks, n_act)
    return pl.kernel(
        body,
        out_type=jax.ShapeDtypeStruct((BATCH, n_act * rw, c), jnp.float32),
        mesh=mesh,
        scratch_types=[
            pltpu.VMEM((chunks, 128), jnp.int32),
            pltpu.VMEM((chunks, 128), jnp.int32),
            pltpu.VMEM((chunks * 128, c), jnp.float32),
            pltpu.SemaphoreType.DMA,
        ],
        compiler_params=pltpu.CompilerParams(use_tc_tiling_on_sc=False),
    )(sic_pad, h2)


def _level_body(m, g_ref, w_ref, b_ref, dv_ref, out_ref):
    g = g_ref[0]
    y = jnp.dot(g, w_ref[...], preferred_element_type=jnp.float32)
    y = y + b_ref[...]
    y = jnp.where(y > 0, y, jnp.exp(y) - 1.0)
    co = y.shape[-1]
    y = y[: 3 * m] * dv_ref[...]
    y = y.reshape(m, 3, co)
    out_ref[0] = jnp.sum(y, axis=1)


def _level_call(g, w, b, dv, m):
    bsz, epad, kdim = g.shape
    co = w.shape[1]
    e = 3 * m
    return pl.pallas_call(
        functools.partial(_level_body, m),
        grid=(bsz,),
        in_specs=[
            pl.BlockSpec((1, epad, kdim), lambda ib: (ib, 0, 0)),
            pl.BlockSpec((kdim, co), lambda ib: (0, 0)),
            pl.BlockSpec((1, co), lambda ib: (0, 0)),
            pl.BlockSpec((e, 1), lambda ib: (0, 0)),
        ],
        out_specs=pl.BlockSpec((1, m, co), lambda ib: (ib, 0, 0)),
        out_shape=jax.ShapeDtypeStruct((bsz, m, co), jnp.float32),
    )(g, w, b.reshape(1, co), dv.reshape(e, 1))


def _head_body(h_ref, wlat_ref, blat_ref, wcls_ref, bcls_ref, out_ref):
    h = h_ref[...]
    z = jnp.dot(h, wlat_ref[...], preferred_element_type=jnp.float32)
    z = z + blat_ref[...]
    out_ref[...] = jnp.dot(z, wcls_ref[...], preferred_element_type=jnp.float32) + bcls_ref[...]


def _head_call(h, wlat, blat, wcls, bcls):
    bsz = h.shape[0]
    return pl.pallas_call(
        _head_body,
        out_shape=jax.ShapeDtypeStruct((bsz, NUM_OUT), jnp.float32),
    )(h, wlat, blat.reshape(1, LATENT), wcls, bcls.reshape(1, NUM_OUT))


def kernel(x, si0, si1, si2, si3, dr0, dr1, dr2, dr3, dc0, dc1, dc2, dc3,
           dv0, dv1, dv2, dv3, W0, b0, W1, b1, W2, b2, W3, b3,
           Wlat, blat, Wcls, bcls):
    si = (si0, si1, si2, si3)
    dc = (dc0, dc1, dc2, dc3)
    dv = (dv0, dv1, dv2, dv3)
    Ws = (W0, W1, W2, W3)
    bs = (b0, b1, b2, b3)
    h = x
    for i in range(4):
        m = LEVELS[i + 1]
        n_in = LEVELS[i]
        c, cg = CH[i], CG[i]
        rw, chunks, n_act = RW[i], CHUNKS[i], N_ACT[i]
        sic = jnp.take(si[i], dc[i], axis=0).reshape(-1).astype(jnp.int32)
        sic = jnp.pad(sic, (0, n_act * rw - sic.shape[0])).reshape(n_act, rw)
        sic = jnp.pad(sic, ((0, 0), (0, chunks * 128 - rw)))
        sic_pad = sic.reshape(n_act, chunks, 128)
        w = Ws[i]
        if cg != c:
            h = jnp.pad(h, ((0, 0), (0, 0), (0, cg - c)))
            w = jnp.pad(w.reshape(SPIRAL, c, -1),
                        ((0, 0), (0, cg - c), (0, 0))).reshape(SPIRAL * cg, -1)
        h2 = h.reshape(BATCH * n_in, cg)
        g = _sc_gather(h2, sic_pad, n_in, cg, i)
        g = g.reshape(BATCH, EPAD[i], SPIRAL * cg)
        h = _level_call(g, w, bs[i], dv[i], m)
    hflat = h.reshape(BATCH, LEVELS[4] * CH[4])
    return _head_call(hflat, Wlat, blat, Wcls, bcls)


# speedup vs baseline: 1.3792x; 1.3792x over previous
import functools

import jax
import jax.numpy as jnp
from jax import lax
from jax.experimental import pallas as pl
from jax.experimental.pallas import tpu as pltpu
from jax.experimental.pallas import tpu_sc as plsc

LEVELS = [10000, 2500, 625, 160, 40]
CH = [3, 32, 64, 128, 256]
CG = [8, 32, 64, 128]
SPIRAL = 9
BATCH = 64
LATENT = 256
NUM_OUT = 10

NW = 32
N_ACT = [32, 32, 32, 8]
RW = [2160, 576, 144, 144]
EPAD = [n * r // SPIRAL for n, r in zip(N_ACT, RW)]
CHUNKS = [(r + 127) // 128 for r in RW]


def _gather_body(n_in, rw, chunks, n_act, sic_ref, h_ref, g_ref,
                 idxr, idxb0, idxb1, st0, st1, semg0, semg1, semo0, semo1):
    wid = lax.axis_index("s") * 2 + lax.axis_index("c")
    e9 = n_act * rw

    @pl.when(wid < n_act)
    def _():
        pltpu.sync_copy(sic_ref.at[wid], idxr)
        idxbs = (idxb0, idxb1)
        stages = (st0, st1)
        semgs = (semg0, semg1)
        semos = (semo0, semo1)

        def fire(b, p):
            base = b * n_in
            for j in range(chunks):
                for k in range(8):
                    sl = pl.ds(k * 16, 16)
                    idxbs[p][j, sl] = idxr[j, sl] + base
            handles = []
            for j in range(chunks):
                handles.append(
                    pltpu.async_copy(
                        h_ref.at[idxbs[p].at[j]],
                        stages[p].at[pl.ds(j * 128, 128)],
                        semgs[p],
                    )
                )
            return handles

        def drain_out(b, p):
            pltpu.make_async_copy(
                stages[p].at[pl.ds(0, rw)],
                g_ref.at[pl.ds(b * e9 + wid * rw, rw)],
                semos[p],
            ).wait()

        def start_out(b, p):
            pltpu.async_copy(
                stages[p].at[pl.ds(0, rw)],
                g_ref.at[pl.ds(b * e9 + wid * rw, rw)],
                semos[p],
            )

        def b_loop(bb, carry):
            for p in (0, 1):
                b = 2 * bb + p

                @pl.when(bb > 0)
                def _():
                    drain_out(b - 2, p)

                for h in fire(b, p):
                    h.wait()
                start_out(b, p)
            return carry

        lax.fori_loop(0, BATCH // 2, b_loop, 0)
        drain_out(BATCH - 2, 0)
        drain_out(BATCH - 1, 1)


def _sc_gather(h2, sic_pad, n_in, c, lvl):
    rw, chunks, n_act = RW[lvl], CHUNKS[lvl], N_ACT[lvl]
    mesh = plsc.VectorSubcoreMesh(core_axis_name="c", subcore_axis_name="s")
    body = functools.partial(_gather_body, n_in, rw, chunks, n_act)
    return pl.kernel(
        body,
        out_type=jax.ShapeDtypeStruct((BATCH * n_act * rw, c), jnp.float32),
        mesh=mesh,
        scratch_types=[
            pltpu.VMEM((chunks, 128), jnp.int32),
            pltpu.VMEM((chunks, 128), jnp.int32),
            pltpu.VMEM((chunks, 128), jnp.int32),
            pltpu.VMEM((chunks * 128, c), jnp.float32),
            pltpu.VMEM((chunks * 128, c), jnp.float32),
            pltpu.SemaphoreType.DMA,
            pltpu.SemaphoreType.DMA,
            pltpu.SemaphoreType.DMA,
            pltpu.SemaphoreType.DMA,
        ],
        compiler_params=pltpu.CompilerParams(use_tc_tiling_on_sc=False),
    )(sic_pad, h2)


def _level_body(m, g_ref, w_ref, b_ref, dv_ref, out_ref):
    g = g_ref[0]
    y = jnp.dot(g, w_ref[...], preferred_element_type=jnp.float32)
    y = y + b_ref[...]
    y = jnp.where(y > 0, y, jnp.exp(y) - 1.0)
    co = y.shape[-1]
    y = y[: 3 * m] * dv_ref[...]
    y = y.reshape(m, 3, co)
    out_ref[0] = jnp.sum(y, axis=1)


def _level_call(g, w, b, dv, m):
    bsz, epad, kdim = g.shape
    co = w.shape[1]
    e = 3 * m
    return pl.pallas_call(
        functools.partial(_level_body, m),
        grid=(bsz,),
        in_specs=[
            pl.BlockSpec((1, epad, kdim), lambda ib: (ib, 0, 0)),
            pl.BlockSpec((kdim, co), lambda ib: (0, 0)),
            pl.BlockSpec((1, co), lambda ib: (0, 0)),
            pl.BlockSpec((e, 1), lambda ib: (0, 0)),
        ],
        out_specs=pl.BlockSpec((1, m, co), lambda ib: (ib, 0, 0)),
        out_shape=jax.ShapeDtypeStruct((bsz, m, co), jnp.float32),
    )(g, w, b.reshape(1, co), dv.reshape(e, 1))


def _head_body(h_ref, wlat_ref, blat_ref, wcls_ref, bcls_ref, out_ref):
    h = h_ref[...]
    z = jnp.dot(h, wlat_ref[...], preferred_element_type=jnp.float32)
    z = z + blat_ref[...]
    out_ref[...] = jnp.dot(z, wcls_ref[...], preferred_element_type=jnp.float32) + bcls_ref[...]


def _head_call(h, wlat, blat, wcls, bcls):
    bsz = h.shape[0]
    return pl.pallas_call(
        _head_body,
        out_shape=jax.ShapeDtypeStruct((bsz, NUM_OUT), jnp.float32),
    )(h, wlat, blat.reshape(1, LATENT), wcls, bcls.reshape(1, NUM_OUT))


def kernel(x, si0, si1, si2, si3, dr0, dr1, dr2, dr3, dc0, dc1, dc2, dc3,
           dv0, dv1, dv2, dv3, W0, b0, W1, b1, W2, b2, W3, b3,
           Wlat, blat, Wcls, bcls):
    si = (si0, si1, si2, si3)
    dc = (dc0, dc1, dc2, dc3)
    dv = (dv0, dv1, dv2, dv3)
    Ws = (W0, W1, W2, W3)
    bs = (b0, b1, b2, b3)
    h = x
    for i in range(4):
        m = LEVELS[i + 1]
        n_in = LEVELS[i]
        c, cg = CH[i], CG[i]
        rw, chunks, n_act = RW[i], CHUNKS[i], N_ACT[i]
        sic = jnp.take(si[i], dc[i], axis=0).reshape(-1).astype(jnp.int32)
        sic = jnp.pad(sic, (0, n_act * rw - sic.shape[0])).reshape(n_act, rw)
        sic = jnp.pad(sic, ((0, 0), (0, chunks * 128 - rw)))
        sic_pad = sic.reshape(n_act, chunks, 128)
        w = Ws[i]
        if cg != c:
            h = jnp.pad(h, ((0, 0), (0, 0), (0, cg - c)))
            w = jnp.pad(w.reshape(SPIRAL, c, -1),
                        ((0, 0), (0, cg - c), (0, 0))).reshape(SPIRAL * cg, -1)
        h2 = h.reshape(BATCH * n_in, cg)
        g = _sc_gather(h2, sic_pad, n_in, cg, i)
        g = g.reshape(BATCH, EPAD[i], SPIRAL * cg)
        h = _level_call(g, w, bs[i], dv[i], m)
    hflat = h.reshape(BATCH, LEVELS[4] * CH[4])
    return _head_call(hflat, Wlat, blat, Wcls, bcls)


# speedup vs baseline: 1.5045x; 1.0909x over previous
import functools

import jax
import jax.numpy as jnp
from jax import lax
from jax.experimental import pallas as pl
from jax.experimental.pallas import tpu as pltpu
from jax.experimental.pallas import tpu_sc as plsc

LEVELS = [10000, 2500, 625, 160, 40]
CH = [3, 32, 64, 128, 256]
CG = [8, 32, 64, 128]
SPIRAL = 9
BATCH = 64
LATENT = 256
NUM_OUT = 10

NW = 32
N_ACT = [32, 32, 32, 8]
RW = [2304, 576, 144, 144]
EPAD = [n * r // SPIRAL for n, r in zip(N_ACT, RW)]
CHUNKS = [(r + 127) // 128 for r in RW]


def _gather_body(n_in, rw, chunks, n_act, c, wout, sic_ref, h_ref, g_ref,
                 idxr, idxb0, idxb1, st0, st1, semg0, semg1, semo0, semo1):
    wid = lax.axis_index("s") * 2 + lax.axis_index("c")
    e9 = n_act * rw

    @pl.when(wid < n_act)
    def _():
        pltpu.sync_copy(sic_ref.at[wid], idxr)
        idxbs = (idxb0, idxb1)
        stages = (st0, st1)
        semgs = (semg0, semg1)
        semos = (semo0, semo1)

        def fire(b, p):
            base = b * n_in
            for j in range(chunks):
                for k in range(8):
                    sl = pl.ds(k * 16, 16)
                    idxbs[p][j, sl] = idxr[j, sl] + base
            handles = []
            for j in range(chunks):
                handles.append(
                    pltpu.async_copy(
                        h_ref.at[idxbs[p].at[j]],
                        stages[p].at[pl.ds(j * 128, 128)],
                        semgs[p],
                    )
                )
            return handles

        def _out_views(b, p):
            return (stages[p].at[pl.ds(0, rw)],
                    g_ref.at[pl.ds(b * e9 + wid * rw, rw)])

        def drain_out(b, p):
            src, dst = _out_views(b, p)
            pltpu.make_async_copy(src, dst, semos[p]).wait()

        def start_out(b, p):
            src, dst = _out_views(b, p)
            pltpu.async_copy(src, dst, semos[p])

        def b_loop(bb, carry):
            handles = [None, None]
            for p in (0, 1):
                b = 2 * bb + p

                @pl.when(bb > 0)
                def _():
                    drain_out(b - 2, p)

                handles[p] = fire(b, p)
            for p in (0, 1):
                for h in handles[p]:
                    h.wait()
                start_out(2 * bb + p, p)
            return carry

        lax.fori_loop(0, BATCH // 2, b_loop, 0)
        drain_out(BATCH - 2, 0)
        drain_out(BATCH - 1, 1)


def _sc_gather(h2, sic_pad, n_in, c, lvl):
    rw, chunks, n_act = RW[lvl], CHUNKS[lvl], N_ACT[lvl]
    mesh = plsc.VectorSubcoreMesh(core_axis_name="c", subcore_axis_name="s")
    body = functools.partial(_gather_body, n_in, rw, chunks, n_act, c, c)
    return pl.kernel(
        body,
        out_type=jax.ShapeDtypeStruct((BATCH * n_act * rw, c), jnp.float32),
        mesh=mesh,
        scratch_types=[
            pltpu.VMEM((chunks, 128), jnp.int32),
            pltpu.VMEM((chunks, 128), jnp.int32),
            pltpu.VMEM((chunks, 128), jnp.int32),
            pltpu.VMEM((chunks * 128, c), jnp.float32),
            pltpu.VMEM((chunks * 128, c), jnp.float32),
            pltpu.SemaphoreType.DMA,
            pltpu.SemaphoreType.DMA,
            pltpu.SemaphoreType.DMA,
            pltpu.SemaphoreType.DMA,
        ],
        compiler_params=pltpu.CompilerParams(use_tc_tiling_on_sc=False),
    )(sic_pad, h2)


def _level_body(m, g_ref, w_ref, b_ref, dv_ref, out_ref):
    g = g_ref[0]
    y = jnp.dot(g, w_ref[...], preferred_element_type=jnp.float32)
    y = y + b_ref[...]
    y = jnp.where(y > 0, y, jnp.exp(y) - 1.0)
    co = y.shape[-1]
    y = y[: 3 * m] * dv_ref[...]
    y = y.reshape(m, 3, co)
    out_ref[0] = jnp.sum(y, axis=1)


def _level_call(g, w, b, dv, m):
    bsz, epad, kdim = g.shape
    co = w.shape[1]
    e = 3 * m
    return pl.pallas_call(
        functools.partial(_level_body, m),
        grid=(bsz,),
        in_specs=[
            pl.BlockSpec((1, epad, kdim), lambda ib: (ib, 0, 0)),
            pl.BlockSpec((kdim, co), lambda ib: (0, 0)),
            pl.BlockSpec((1, co), lambda ib: (0, 0)),
            pl.BlockSpec((e, 1), lambda ib: (0, 0)),
        ],
        out_specs=pl.BlockSpec((1, m, co), lambda ib: (ib, 0, 0)),
        out_shape=jax.ShapeDtypeStruct((bsz, m, co), jnp.float32),
    )(g, w, b.reshape(1, co), dv.reshape(e, 1))


def _head_body(h_ref, wlat_ref, blat_ref, wcls_ref, bcls_ref, out_ref):
    h = h_ref[...]
    z = jnp.dot(h, wlat_ref[...], preferred_element_type=jnp.float32)
    z = z + blat_ref[...]
    out_ref[...] = jnp.dot(z, wcls_ref[...], preferred_element_type=jnp.float32) + bcls_ref[...]


def _head_call(h, wlat, blat, wcls, bcls):
    bsz = h.shape[0]
    return pl.pallas_call(
        _head_body,
        out_shape=jax.ShapeDtypeStruct((bsz, NUM_OUT), jnp.float32),
    )(h, wlat, blat.reshape(1, LATENT), wcls, bcls.reshape(1, NUM_OUT))


def kernel(x, si0, si1, si2, si3, dr0, dr1, dr2, dr3, dc0, dc1, dc2, dc3,
           dv0, dv1, dv2, dv3, W0, b0, W1, b1, W2, b2, W3, b3,
           Wlat, blat, Wcls, bcls):
    si = (si0, si1, si2, si3)
    dc = (dc0, dc1, dc2, dc3)
    dv = (dv0, dv1, dv2, dv3)
    Ws = (W0, W1, W2, W3)
    bs = (b0, b1, b2, b3)
    h = x
    for i in range(4):
        m = LEVELS[i + 1]
        n_in = LEVELS[i]
        c, cg = CH[i], CG[i]
        rw, chunks, n_act = RW[i], CHUNKS[i], N_ACT[i]
        sic = jnp.take(si[i], dc[i], axis=0).reshape(-1).astype(jnp.int32)
        sic = jnp.pad(sic, (0, n_act * rw - sic.shape[0])).reshape(n_act, rw)
        sic = jnp.pad(sic, ((0, 0), (0, chunks * 128 - rw)))
        sic_pad = sic.reshape(n_act, chunks, 128)
        w = Ws[i]
        if cg != c:
            h = jnp.pad(h, ((0, 0), (0, 0), (0, cg - c)))
            w = jnp.pad(w.reshape(SPIRAL, c, -1),
                        ((0, 0), (0, cg - c), (0, 0))).reshape(SPIRAL * cg, -1)
        h2 = h.reshape(BATCH * n_in, cg)
        g = _sc_gather(h2, sic_pad, n_in, cg, i)
        g = g.reshape(BATCH, EPAD[i], SPIRAL * cg)
        h = _level_call(g, w, bs[i], dv[i], m)
    hflat = h.reshape(BATCH, LEVELS[4] * CH[4])
    return _head_call(hflat, Wlat, blat, Wcls, bcls)
